# baseline (speedup 1.0000x reference)
import functools

import jax
import jax.numpy as jnp
from jax import lax
from jax.experimental import pallas as pl
from jax.experimental.pallas import tpu as pltpu
from jax.experimental.pallas import tpu_sc as plsc

B = 1024
T = 50
VOCAB = 1000
NW = 32
B_PER_W = B // NW


def _body(idx_hbm, table_hbm, out_hbm, idx_v, rows_v, sem):
    wid = lax.axis_index("s") * 2 + lax.axis_index("c")
    base = wid * B_PER_W
    pltpu.sync_copy(idx_hbm.at[wid], idx_v)

    def chunk(j, carry):
        pltpu.async_copy(table_hbm.at[idx_v.at[j]], rows_v, sem).wait()
        pltpu.sync_copy(rows_v, out_hbm.at[base + j])
        return carry

    lax.fori_loop(0, B_PER_W, chunk, 0)


@jax.jit
def _gather(idx_grp, table):
    mesh = plsc.VectorSubcoreMesh(core_axis_name="c", subcore_axis_name="s")
    f = functools.partial(
        pl.kernel,
        mesh=mesh,
        out_type=jax.ShapeDtypeStruct((B, T, VOCAB), jnp.float32),
        scratch_types=[
            pltpu.VMEM((B_PER_W, T), jnp.int32),
            pltpu.VMEM((T, VOCAB), jnp.float32),
            pltpu.SemaphoreType.DMA,
        ],
        compiler_params=pltpu.CompilerParams(use_tc_tiling_on_sc=False),
    )(_body)
    return f(idx_grp, table)


def kernel(idx, table):
    idx_grp = idx.reshape(NW, B_PER_W, T)
    return _gather(idx_grp, table)


# speedup vs baseline: 1.3011x; 1.3011x over previous
import functools

import jax
import jax.numpy as jnp
from jax import lax
from jax.experimental import pallas as pl
from jax.experimental.pallas import tpu as pltpu
from jax.experimental.pallas import tpu_sc as plsc

B = 1024
T = 50
VOCAB = 1000
NW = 32
NV8 = VOCAB // 8
NBT = B // 128


def _body(idx_hbm, table_hbm, out_hbm, idx_v, slab_v, buf_v, sem0, sem1):
    wid = lax.axis_index("s") * 2 + lax.axis_index("c")
    cnt = jnp.where(wid < 29, 4, 3)
    start = 4 * wid - jnp.maximum(wid - 29, 0)
    pltpu.sync_copy(idx_hbm, idx_v)

    def do_slab(k, carry):
        v8 = start + k
        pltpu.sync_copy(table_hbm.at[pl.ds(v8 * 8, 8)], slab_v)

        def do_pair(p, carry2):
            for par, sem in ((0, sem0), (1, sem1)):
                t = 2 * p + par

                @pl.when(p >= 1)
                def _wait():
                    pltpu.make_async_copy(
                        buf_v.at[par], out_hbm.at[0, 0], sem).wait()

                def do_bg(bg, carry3):
                    idxv = idx_v[t, pl.ds(bg * 16, 16)]
                    bt = bg >> 3
                    g = bg & 7
                    for vv in range(8):
                        row = jnp.full((16,), vv, jnp.int32)
                        val = plsc.load_gather(slab_v, [row, idxv])
                        buf_v[par, bt, vv, pl.ds(g * 16, 16)] = val
                    return carry3

                lax.fori_loop(0, 64, do_bg, 0)
                pltpu.async_copy(buf_v.at[par], out_hbm.at[t, v8], sem)
            return carry2

        lax.fori_loop(0, T // 2, do_pair, 0)
        pltpu.make_async_copy(buf_v.at[0], out_hbm.at[0, 0], sem0).wait()
        pltpu.make_async_copy(buf_v.at[1], out_hbm.at[0, 0], sem1).wait()
        return carry

    lax.fori_loop(0, cnt, do_slab, 0)


@jax.jit
def _gather(idx_t, table_t):
    mesh = plsc.VectorSubcoreMesh(core_axis_name="c", subcore_axis_name="s")
    f = functools.partial(
        pl.kernel,
        mesh=mesh,
        out_type=jax.ShapeDtypeStruct((T, NV8, NBT, 8, 128), jnp.float32),
        scratch_types=[
            pltpu.VMEM((T, B), jnp.int32),
            pltpu.VMEM((8, VOCAB), jnp.float32),
            pltpu.VMEM((2, NBT, 8, 128), jnp.float32),
            pltpu.SemaphoreType.DMA,
            pltpu.SemaphoreType.DMA,
        ],
        compiler_params=pltpu.CompilerParams(use_tc_tiling_on_sc=False, needs_layout_passes=False),
    )(_body)
    return f(idx_t, table_t)


def kernel(idx, table):
    out5 = _gather(idx.T, table.T)
    return out5.transpose(2, 4, 0, 1, 3).reshape(B, T, VOCAB)


# speedup vs baseline: 6.6982x; 5.1479x over previous
import functools

import jax
import jax.numpy as jnp
from jax import lax
from jax.experimental import pallas as pl
from jax.experimental.pallas import tpu as pltpu
from jax.experimental.pallas import tpu_sc as plsc

B = 1024
T = 50
VOCAB = 1000
NW = 32
NV8 = VOCAB // 8
NBT = B // 128


def _body(idx_hbm, table_hbm, out_hbm, idx_v, slab_v, buf_v, sem0, sem1):
    wid = lax.axis_index("s") * 2 + lax.axis_index("c")
    cnt = jnp.where(wid < 29, 4, 3)
    start = 4 * wid - jnp.maximum(wid - 29, 0)
    pltpu.sync_copy(idx_hbm, idx_v)

    def do_slab(k, carry):
        v8 = start + k
        pltpu.sync_copy(table_hbm.at[pl.ds(v8 * 8, 8)], slab_v)

        def do_pair(p, carry2):
            for par, sem in ((0, sem0), (1, sem1)):
                t = 2 * p + par

                @pl.when(p >= 1)
                def _wait():
                    pltpu.make_async_copy(
                        buf_v.at[par], out_hbm.at[0, 0], sem).wait()

                @functools.partial(plsc.parallel_loop, 0, NBT)
                def _bt_loop(bt):
                    for g in range(8):
                        idxv = idx_v[t, pl.ds(bt * 128 + g * 16, 16)]
                        for vv in range(8):
                            row = jnp.full((16,), vv, jnp.int32)
                            val = plsc.load_gather(slab_v, [row, idxv])
                            buf_v[par, bt, vv, pl.ds(g * 16, 16)] = val
                pltpu.async_copy(buf_v.at[par], out_hbm.at[t, v8], sem)
            return carry2

        lax.fori_loop(0, T // 2, do_pair, 0)
        pltpu.make_async_copy(buf_v.at[0], out_hbm.at[0, 0], sem0).wait()
        pltpu.make_async_copy(buf_v.at[1], out_hbm.at[0, 0], sem1).wait()
        return carry

    lax.fori_loop(0, cnt, do_slab, 0)


@jax.jit
def _gather(idx_t, table_t):
    mesh = plsc.VectorSubcoreMesh(core_axis_name="c", subcore_axis_name="s")
    f = functools.partial(
        pl.kernel,
        mesh=mesh,
        out_type=jax.ShapeDtypeStruct((T, NV8, NBT, 8, 128), jnp.float32),
        scratch_types=[
            pltpu.VMEM((T, B), jnp.int32),
            pltpu.VMEM((8, VOCAB), jnp.float32),
            pltpu.VMEM((2, NBT, 8, 128), jnp.float32),
            pltpu.SemaphoreType.DMA,
            pltpu.SemaphoreType.DMA,
        ],
        compiler_params=pltpu.CompilerParams(use_tc_tiling_on_sc=False, needs_layout_passes=False),
    )(_body)
    return f(idx_t, table_t)


def kernel(idx, table):
    out5 = _gather(idx.T, table.T)
    return out5.transpose(2, 4, 0, 1, 3).reshape(B, T, VOCAB)


# speedup vs baseline: 6.8988x; 1.0300x over previous
import functools

import jax
import jax.numpy as jnp
from jax import lax
from jax.experimental import pallas as pl
from jax.experimental.pallas import tpu as pltpu
from jax.experimental.pallas import tpu_sc as plsc

B = 1024
T = 50
VOCAB = 1000
NW = 32
NV8 = VOCAB // 8
NBT = B // 128


def _body(idx_hbm, table_hbm, out_hbm, idx_v, slab_v, buf_v,
          sem0, sem1, ssem0, ssem1, isem):
    wid = lax.axis_index("s") * 2 + lax.axis_index("c")
    cnt = jnp.where(wid < 29, 4, 3)
    start = 4 * wid - jnp.maximum(wid - 29, 0)
    ssems = (ssem0, ssem1)
    pltpu.async_copy(idx_hbm, idx_v, isem)
    pltpu.async_copy(table_hbm.at[pl.ds(start * 8, 8)], slab_v.at[0], ssem0)
    pltpu.make_async_copy(idx_hbm, idx_v, isem).wait()

    for k in range(4):
        sb = k & 1

        @pl.when(k < cnt)
        def _slab():
            v8 = start + k
            pltpu.make_async_copy(
                table_hbm.at[pl.ds(0, 8)], slab_v.at[sb], ssems[sb]).wait()

            @pl.when(k + 1 < cnt)
            def _prefetch():
                pltpu.async_copy(table_hbm.at[pl.ds((v8 + 1) * 8, 8)],
                                 slab_v.at[1 - sb], ssems[1 - sb])

            def do_pair(p, carry2):
                for par, sem in ((0, sem0), (1, sem1)):
                    t = 2 * p + par

                    @pl.when((k > 0) | (p >= 1))
                    def _wait():
                        pltpu.make_async_copy(
                            buf_v.at[par], out_hbm.at[0, 0], sem).wait()

                    @functools.partial(plsc.parallel_loop, 0, NBT)
                    def _bt_loop(bt):
                        for g in range(8):
                            idxv = idx_v[t, pl.ds(bt * 128 + g * 16, 16)]
                            for vv in range(8):
                                row = jnp.full((16,), vv, jnp.int32)
                                val = plsc.load_gather(slab_v.at[sb],
                                                       [row, idxv])
                                buf_v[par, bt, vv, pl.ds(g * 16, 16)] = val
                    pltpu.async_copy(buf_v.at[par], out_hbm.at[t, v8], sem)
                return carry2

            lax.fori_loop(0, T // 2, do_pair, 0)

    pltpu.make_async_copy(buf_v.at[0], out_hbm.at[0, 0], sem0).wait()
    pltpu.make_async_copy(buf_v.at[1], out_hbm.at[0, 0], sem1).wait()


@jax.jit
def _gather(idx_t, table_t):
    mesh = plsc.VectorSubcoreMesh(core_axis_name="c", subcore_axis_name="s")
    f = functools.partial(
        pl.kernel,
        mesh=mesh,
        out_type=jax.ShapeDtypeStruct((T, NV8, NBT, 8, 128), jnp.float32),
        scratch_types=[
            pltpu.VMEM((T, B), jnp.int32),
            pltpu.VMEM((2, 8, VOCAB), jnp.float32),
            pltpu.VMEM((2, NBT, 8, 128), jnp.float32),
            pltpu.SemaphoreType.DMA,
            pltpu.SemaphoreType.DMA,
            pltpu.SemaphoreType.DMA,
            pltpu.SemaphoreType.DMA,
            pltpu.SemaphoreType.DMA,
        ],
        compiler_params=pltpu.CompilerParams(use_tc_tiling_on_sc=False, needs_layout_passes=False),
    )(_body)
    return f(idx_t, table_t)


def kernel(idx, table):
    out5 = _gather(idx.T, table.T)
    return out5.transpose(2, 4, 0, 1, 3).reshape(B, T, VOCAB)
